# Initial kernel scaffold; baseline (speedup 1.0000x reference)
import jax
import jax.numpy as jnp
from jax.experimental import pallas as pl


def kernel(inputs, emb_w):
    raise NotImplementedError("write your pallas kernel here")


# speedup vs baseline: 1.1139x; 1.1139x over previous
import jax
import jax.numpy as jnp
from jax.experimental import pallas as pl
from jax.experimental.pallas import tpu as pltpu

NCODES = 1000
CPAD = 1024
DIM = 20
TB = 512


def _vq_body(x_ref, ewp_ref, ewt_ref, e2_ref, q_ref, idx_ref, hist_ref, loss_ref):
    step = pl.program_id(0)
    xb = x_ref[0]

    dot = jax.lax.dot_general(
        ewp_ref[...], xb, (((1,), (0,)), ((), ())),
        preferred_element_type=jnp.float32)
    dist = e2_ref[...] - 2.0 * dot

    minv = jnp.min(dist, axis=0, keepdims=True)
    riota = jax.lax.broadcasted_iota(jnp.int32, (CPAD, TB), 0)
    idx = jnp.min(jnp.where(dist <= minv, riota, CPAD), axis=0)
    onehot = jnp.where(riota == idx[None, :], 1.0, 0.0)

    qT = jax.lax.dot_general(
        ewt_ref[...], onehot, (((1,), (0,)), ((), ())),
        preferred_element_type=jnp.float32)
    q_ref[0] = qT
    idx_ref[0, 0] = idx

    h = jnp.sum(onehot, axis=1, keepdims=True)

    mx_i = jnp.max(xb, axis=0, keepdims=True)
    ex = jnp.exp(xb - mx_i)
    se = jnp.sum(ex, axis=0, keepdims=True)
    sm_i = ex / se
    log_sm_i = (xb - mx_i) - jnp.log(se)
    mx_q = jnp.max(qT, axis=0, keepdims=True)
    eq = jnp.exp(qT - mx_q)
    sm_q = eq / jnp.sum(eq, axis=0, keepdims=True)
    tile_loss = jnp.sum(sm_i * (log_sm_i - sm_q)).reshape(1, 1)

    @pl.when(step == 0)
    def _init():
        hist_ref[...] = jnp.zeros_like(hist_ref)
        loss_ref[...] = jnp.zeros_like(loss_ref)

    hist_ref[...] += h
    loss_ref[...] += tile_loss


def kernel(inputs, emb_w):
    B, C, T = inputs.shape
    nt = T // TB
    ng = B * nt
    ewp = jnp.zeros((CPAD, DIM), jnp.float32).at[:NCODES].set(emb_w)
    e2 = jnp.full((CPAD, 1), 1e30, jnp.float32).at[:NCODES, 0].set(
        jnp.sum(emb_w * emb_w, axis=1))
    ewt = ewp.T

    q, idxo, hist, loss_sum = pl.pallas_call(
        _vq_body,
        grid=(ng,),
        in_specs=[
            pl.BlockSpec((1, DIM, TB), lambda g: (g // nt, 0, g % nt)),
            pl.BlockSpec((CPAD, DIM), lambda g: (0, 0)),
            pl.BlockSpec((DIM, CPAD), lambda g: (0, 0)),
            pl.BlockSpec((CPAD, 1), lambda g: (0, 0)),
        ],
        out_specs=[
            pl.BlockSpec((1, DIM, TB), lambda g: (g // nt, 0, g % nt)),
            pl.BlockSpec((1, 1, TB), lambda g: (g, 0, 0)),
            pl.BlockSpec((CPAD, 1), lambda g: (0, 0)),
            pl.BlockSpec((1, 1), lambda g: (0, 0)),
        ],
        out_shape=[
            jax.ShapeDtypeStruct((B, C, T), jnp.float32),
            jax.ShapeDtypeStruct((ng, 1, TB), jnp.int32),
            jax.ShapeDtypeStruct((CPAD, 1), jnp.float32),
            jax.ShapeDtypeStruct((1, 1), jnp.float32),
        ],
        compiler_params=pltpu.CompilerParams(
            dimension_semantics=("arbitrary",)),
    )(inputs, ewp, ewt, e2)

    enc_idx = idxo.reshape(-1)
    n = B * T
    avg = hist[:NCODES, 0] / n
    perplexity = jnp.exp(-jnp.sum(avg * jnp.log(avg + 1e-10)))
    loss = 0.1 * loss_sum[0, 0] / B
    return q, loss, perplexity, emb_w, enc_idx
